# baseline (speedup 1.0000x reference)
import functools

import jax
import jax.numpy as jnp
from jax import lax
from jax.experimental import pallas as pl
from jax.experimental.pallas import tpu as pltpu
from jax.experimental.pallas import tpu_sc as plsc

_N = 1000
_NPAD = 1024
_NCHUNK = _NPAD // 16
_K = 16
_DOUT = 64
_FEAT = 18
_BATCH = 32
_NBLK = 8
_ROWS_PER_BLK = _N // _NBLK


def _merge16(ak, av, bk, bv):
    rk = lax.rev(bk, (0,))
    rv = lax.rev(bv, (0,))
    m = ak <= rk
    nk = jnp.where(m, ak, rk)
    nv = jnp.where(m, av, rv)
    nk, nv = plsc.sort_key_val(nk, nv)
    return nk, nv


def _mlp_tc_kernel(x_ref, w_ref, b_ref, y_ref):
    y_ref[...] = (
        jnp.dot(x_ref[...], w_ref[...], preferred_element_type=jnp.float32)
        + b_ref[...]
    )


def _edge_sc_body(x_hbm, y_hbm, a_hbm, b_hbm, out_hbm, xv, yv, bv, av, ov, iv):
    wid = lax.axis_index("s") * 2 + lax.axis_index("c")
    pltpu.sync_copy(x_hbm.at[wid], xv)
    pltpu.sync_copy(y_hbm.at[wid], yv)
    pltpu.sync_copy(b_hbm.at[wid], bv)

    iota = lax.broadcasted_iota(jnp.int32, (16,), 0)
    inf = jnp.float32(jnp.inf)
    init_k = jnp.full((16,), inf, jnp.float32)
    init_v = jnp.zeros((16,), jnp.int32)

    for blk in range(_NBLK):
        pltpu.sync_copy(
            a_hbm.at[wid, pl.ds(blk * _ROWS_PER_BLK * _DOUT,
                                _ROWS_PER_BLK * _DOUT)], av)

        def row_body(r, _, blk=blk):
            i = blk * _ROWS_PER_BLK + r
            ii = jnp.full((16,), i, jnp.int32)
            xi = plsc.load_gather(xv, [ii])
            yi = plsc.load_gather(yv, [ii])

            def oct_body(c, carry):
                lk, lv = carry
                leaves = []
                for q in range(8):
                    base = c * 128 + q * 16
                    dx = xv[pl.ds(base, 16)] - xi
                    dy = yv[pl.ds(base, 16)] - yi
                    d = dx * dx + dy * dy
                    jc = iota + base
                    d = jnp.where(jc == i, inf, d)
                    leaves.append(plsc.sort_key_val(d, jc))
                m1 = _merge16(*leaves[0], *leaves[1])
                m2 = _merge16(*leaves[2], *leaves[3])
                m3 = _merge16(*leaves[4], *leaves[5])
                m4 = _merge16(*leaves[6], *leaves[7])
                n1 = _merge16(*m1, *m2)
                n2 = _merge16(*m3, *m4)
                n3 = _merge16(*n1, *n2)
                return _merge16(lk, lv, *n3)

            _, lv = lax.fori_loop(0, _NCHUNK // 8, oct_body,
                                  (init_k, init_v))
            iv[pl.ds(16, 16)] = lv

            a = [av[pl.ds(r * _DOUT + 16 * c4, 16)] for c4 in range(4)]
            acc = [jnp.zeros((16,), jnp.float32) for _ in range(4)]
            for k in range(_K):
                nb = plsc.load_gather(iv, [jnp.full((16,), 16 + k, jnp.int32)])
                for c4 in range(4):
                    col = iota + 16 * c4
                    bvec = plsc.load_gather(bv, [nb, col])
                    acc[c4] = acc[c4] + jnp.maximum(bvec + a[c4], 0.0)
            scale = jnp.float32(1.0 / _K)
            for c4 in range(4):
                ov[pl.ds(r * _DOUT + 16 * c4, 16)] = acc[c4] * scale
            return 0

        lax.fori_loop(0, _ROWS_PER_BLK, row_body, 0)
        pltpu.sync_copy(
            ov, out_hbm.at[wid, pl.ds(blk * _ROWS_PER_BLK * _DOUT,
                                      _ROWS_PER_BLK * _DOUT)])


@functools.partial(
    pl.kernel,
    out_type=jax.ShapeDtypeStruct((_BATCH, _N * _DOUT), jnp.float32),
    mesh=plsc.VectorSubcoreMesh(core_axis_name="c", subcore_axis_name="s"),
    compiler_params=pltpu.CompilerParams(
        needs_layout_passes=False, use_tc_tiling_on_sc=False),
    scratch_types=[
        pltpu.VMEM((_NPAD,), jnp.float32),
        pltpu.VMEM((_NPAD,), jnp.float32),
        pltpu.VMEM((_N, _DOUT), jnp.float32),
        pltpu.VMEM((_ROWS_PER_BLK * _DOUT,), jnp.float32),
        pltpu.VMEM((_ROWS_PER_BLK * _DOUT,), jnp.float32),
        pltpu.VMEM((32,), jnp.int32),
    ],
)
def _edge_sc(x_hbm, y_hbm, a_hbm, b_hbm, out_hbm, xv, yv, bv, av, ov, iv):
    _edge_sc_body(x_hbm, y_hbm, a_hbm, b_hbm, out_hbm, xv, yv, bv, av, ov, iv)


def kernel(inputs, W, b):
    x = inputs[:, :, 0]
    y = inputs[:, :, 1]
    pad = jnp.full((_BATCH, _NPAD - _N), 1e30, jnp.float32)
    xp = jnp.concatenate([x, pad], axis=1)
    yp = jnp.concatenate([y, pad], axis=1)

    xf = inputs[:, :, :_FEAT].reshape(_BATCH * _N, _FEAT)
    w1 = W[:_FEAT]
    w2 = W[_FEAT:]
    wcat = jnp.concatenate([w1 - w2, w2], axis=1)
    bcat = jnp.concatenate([b, jnp.zeros((_DOUT,), jnp.float32)])[None, :]

    rows_blk = (_BATCH * _N) // 8
    yab = pl.pallas_call(
        _mlp_tc_kernel,
        grid=(8,),
        in_specs=[
            pl.BlockSpec((rows_blk, _FEAT), lambda i: (i, 0)),
            pl.BlockSpec((_FEAT, 2 * _DOUT), lambda i: (0, 0)),
            pl.BlockSpec((1, 2 * _DOUT), lambda i: (0, 0)),
        ],
        out_specs=pl.BlockSpec((rows_blk, 2 * _DOUT), lambda i: (i, 0)),
        out_shape=jax.ShapeDtypeStruct((_BATCH * _N, 2 * _DOUT), jnp.float32),
    )(xf, wcat, bcat)

    a_flat = yab[:, :_DOUT].reshape(_BATCH, _N * _DOUT)
    b_tab = yab[:, _DOUT:].reshape(_BATCH, _N, _DOUT)

    out_flat = _edge_sc(xp, yp, a_flat, b_tab)
    avg = out_flat.reshape(_BATCH, _N, _DOUT)
    ones = jnp.ones((_BATCH, _N, 1), jnp.float32)
    return jnp.concatenate([avg, ones], axis=2)


# speedup vs baseline: 1.0220x; 1.0220x over previous
import functools

import jax
import jax.numpy as jnp
from jax import lax
from jax.experimental import pallas as pl
from jax.experimental.pallas import tpu as pltpu
from jax.experimental.pallas import tpu_sc as plsc

_N = 1000
_NPAD = 1024
_NCHUNK = _NPAD // 16
_K = 16
_DOUT = 64
_DROW = 65
_FEAT = 18
_BATCH = 32
_NBLK = 5
_RPB = _N // _NBLK


def _merge16(ak, av, bk, bv):
    rk = lax.rev(bk, (0,))
    rv = lax.rev(bv, (0,))
    m = ak <= rk
    nk = jnp.where(m, ak, rk)
    nv = jnp.where(m, av, rv)
    nk, nv = plsc.sort_key_val(nk, nv)
    return nk, nv


def _mlp_tc_kernel(in_ref, wd_ref, w2_ref, b_ref, a_ref, bt_ref):
    x = in_ref[0][:, :_FEAT]
    a_ref[...] = (
        jnp.dot(x, wd_ref[...], preferred_element_type=jnp.float32)
        + b_ref[...]
    )[None]
    bt_ref[...] = jnp.dot(
        x, w2_ref[...], preferred_element_type=jnp.float32)[None]


def _edge_sc_body(x_hbm, y_hbm, a_hbm, b_hbm, out_hbm, xv, yv, bv, av, ov, iv):
    wid = lax.axis_index("s") * 2 + lax.axis_index("c")
    pltpu.sync_copy(x_hbm.at[wid], xv)
    pltpu.sync_copy(y_hbm.at[wid], yv)
    pltpu.sync_copy(b_hbm.at[wid], bv)

    iota = lax.broadcasted_iota(jnp.int32, (16,), 0)
    inf = jnp.float32(jnp.inf)
    init_k = jnp.full((16,), inf, jnp.float32)
    init_v = jnp.zeros((16,), jnp.int32)
    onev = jnp.ones((16,), jnp.float32)

    for blk in range(_NBLK):
        pltpu.sync_copy(
            a_hbm.at[wid, pl.ds(blk * _RPB * _DOUT, _RPB * _DOUT)], av)

        def row_body(r, _, blk=blk):
            i = blk * _RPB + r
            ii = jnp.full((16,), i, jnp.int32)
            xi = plsc.load_gather(xv, [ii])
            yi = plsc.load_gather(yv, [ii])

            def quad_body(c, carry):
                lk, lv = carry
                leaves = []
                for q in range(4):
                    base = c * 64 + q * 16
                    dx = xv[pl.ds(base, 16)] - xi
                    dy = yv[pl.ds(base, 16)] - yi
                    d = dx * dx + dy * dy
                    jc = iota + base
                    d = jnp.where(jc == i, inf, d)
                    leaves.append(plsc.sort_key_val(d, jc))
                m1 = _merge16(*leaves[0], *leaves[1])
                m2 = _merge16(*leaves[2], *leaves[3])
                m3 = _merge16(*m1, *m2)
                return _merge16(lk, lv, *m3)

            _, lv = lax.fori_loop(0, _NCHUNK // 4, quad_body,
                                  (init_k, init_v))
            iv[pl.ds(16, 16)] = lv

            a = [av[pl.ds(r * _DOUT + 16 * c4, 16)] for c4 in range(4)]
            acc = [jnp.zeros((16,), jnp.float32) for _ in range(4)]
            for k in range(_K):
                nb = plsc.load_gather(iv, [jnp.full((16,), 16 + k, jnp.int32)])
                for c4 in range(4):
                    col = iota + 16 * c4
                    bvec = plsc.load_gather(bv, [nb, col])
                    acc[c4] = acc[c4] + jnp.maximum(bvec + a[c4], 0.0)
            scale = jnp.float32(1.0 / _K)
            for c4 in range(4):
                ov[pl.ds(r * _DROW + 16 * c4, 16)] = acc[c4] * scale
            return 0

        lax.fori_loop(0, _RPB, row_body, 0)
        last = _RPB * _DROW - 1
        for t in range((_RPB + 15) // 16):
            idx = jnp.minimum(iota * _DROW + _DOUT + t * 16 * _DROW, last)
            plsc.store_scatter(ov, [idx], onev)
        pltpu.sync_copy(
            ov, out_hbm.at[wid, pl.ds(blk * _RPB * _DROW, _RPB * _DROW)])


@functools.partial(
    pl.kernel,
    out_type=jax.ShapeDtypeStruct((_BATCH, _N * _DROW), jnp.float32),
    mesh=plsc.VectorSubcoreMesh(core_axis_name="c", subcore_axis_name="s"),
    compiler_params=pltpu.CompilerParams(
        needs_layout_passes=False, use_tc_tiling_on_sc=False),
    scratch_types=[
        pltpu.VMEM((_NPAD,), jnp.float32),
        pltpu.VMEM((_NPAD,), jnp.float32),
        pltpu.VMEM((_N, _DOUT), jnp.float32),
        pltpu.VMEM((_RPB * _DOUT,), jnp.float32),
        pltpu.VMEM((_RPB * _DROW,), jnp.float32),
        pltpu.VMEM((32,), jnp.int32),
    ],
)
def _edge_sc(x_hbm, y_hbm, a_hbm, b_hbm, out_hbm, xv, yv, bv, av, ov, iv):
    _edge_sc_body(x_hbm, y_hbm, a_hbm, b_hbm, out_hbm, xv, yv, bv, av, ov, iv)


def kernel(inputs, W, b):
    x = inputs[:, :, 0]
    y = inputs[:, :, 1]
    pad = jnp.full((_BATCH, _NPAD - _N), 1e30, jnp.float32)
    xp = jnp.concatenate([x, pad], axis=1)
    yp = jnp.concatenate([y, pad], axis=1)

    w1 = W[:_FEAT]
    w2 = W[_FEAT:]
    wd = w1 - w2
    b2d = b[None, :]

    a_tab, b_tab = pl.pallas_call(
        _mlp_tc_kernel,
        grid=(_BATCH,),
        in_specs=[
            pl.BlockSpec((1, _N, _FEAT + 1), lambda i: (i, 0, 0)),
            pl.BlockSpec((_FEAT, _DOUT), lambda i: (0, 0)),
            pl.BlockSpec((_FEAT, _DOUT), lambda i: (0, 0)),
            pl.BlockSpec((1, _DOUT), lambda i: (0, 0)),
        ],
        out_specs=[
            pl.BlockSpec((1, _N, _DOUT), lambda i: (i, 0, 0)),
            pl.BlockSpec((1, _N, _DOUT), lambda i: (i, 0, 0)),
        ],
        out_shape=[
            jax.ShapeDtypeStruct((_BATCH, _N, _DOUT), jnp.float32),
            jax.ShapeDtypeStruct((_BATCH, _N, _DOUT), jnp.float32),
        ],
    )(inputs, wd, w2, b2d)

    a_flat = a_tab.reshape(_BATCH, _N * _DOUT)
    out_flat = _edge_sc(xp, yp, a_flat, b_tab)
    return out_flat.reshape(_BATCH, _N, _DROW)
